# baseline (speedup 1.0000x reference)
import jax
import jax.numpy as jnp
import numpy as np
from jax import lax
from jax.experimental import pallas as pl
from jax.experimental.pallas import tpu as pltpu
from jax.experimental.pallas import tpu_sc as plsc

_N = 10000
_E = 160000
_NP = 10240
_HW = 128
_NSC = 2
_NTILE = 16
_RPT = _NP // _NTILE
_RPT15 = _N - (_NTILE - 1) * _RPT
_EPT = _E // _NTILE
_C = 80
_NCH = _EPT // _C
_CEPT = _E // (_NSC * _NTILE)
_CC = 128


def _zero_acc(zbuf, acc, r0, s):
    zero16 = jnp.zeros((16,), jnp.float32)

    def zrow(i, carry):
        for j in range(_HW // 16):
            zbuf[i, pl.ds(j * 16, 16)] = zero16
        return carry

    lax.fori_loop(0, 16, zrow, 0)

    def zacc(b, carry):
        pltpu.sync_copy(zbuf, acc.at[pl.ds(r0 + b * 16, 16), :])
        return carry

    lax.fori_loop(0, jnp.where(s == _NTILE - 1, _RPT15 // 16, _RPT // 16),
                  zacc, 0)


def _writeback(acc, out, c, s, r0):
    @pl.when(s < _NTILE - 1)
    def _():
        pltpu.sync_copy(acc.at[pl.ds(r0, _RPT), :],
                        out.at[c, pl.ds(r0, _RPT), :])

    @pl.when(s == _NTILE - 1)
    def _():
        pltpu.sync_copy(acc.at[pl.ds(r0, _RPT15), :],
                        out.at[c, pl.ds(r0, _RPT15), :])


def _agg_body(x2, srcr, dstr, sums_out,
              idx2v0, idx2v1, dstv0, dstv1, dstv2, dstv3,
              rowsb0, rowsb1, rowsv0, rowsv1, zbuf, acc, gsem, ssem):
    c = lax.axis_index("c")
    s = lax.axis_index("s")
    r0 = s * _RPT

    _zero_acc(zbuf, acc, r0, s)
    plsc.subcore_barrier()

    idx2v = (idx2v0, idx2v1)
    dstv = (dstv0, dstv1, dstv2, dstv3)
    rowsb = (rowsb0, rowsb1)
    rowsv = (rowsv0, rowsv1)

    e0 = s * _EPT

    def stage(k, b2, b4):
        eo = e0 + k * _C
        pltpu.sync_copy(srcr.at[pl.ds(eo, _C)], idx2v[b2])
        pltpu.sync_copy(dstr.at[pl.ds(eo, _C)], dstv[b4])
        for j in range(_C // 16):
            v = idx2v[b2][pl.ds(j * 16, 16)]
            idx2v[b2][pl.ds(j * 16, 16)] = v * 2 + c

    def fire_gather(b2):
        pltpu.async_copy(x2.at[idx2v[b2]], rowsb[b2], gsem)

    def wait_gather(b2):
        pltpu.make_async_copy(x2.at[idx2v[b2]], rowsb[b2], gsem).wait()

    himask = jnp.int32(-65536)

    def conv(b2):
        def crow(r, carry):
            for j in range(_HW // 32):
                w = rowsb[b2][r, pl.ds(16 * j, 16)]
                lo = lax.bitcast_convert_type(jnp.left_shift(w, 16),
                                              jnp.float32)
                hi = lax.bitcast_convert_type(jnp.bitwise_and(w, himask),
                                              jnp.float32)
                rowsv[b2][r, pl.ds(32 * j, 16)] = lo
                rowsv[b2][r, pl.ds(32 * j + 16, 16)] = hi
            return carry

        lax.fori_loop(0, _C, crow, 0)

    def fire_scatter(b2, b4):
        pltpu.async_copy(rowsv[b2], acc.at[dstv[b4]], ssem, add=True)

    def wait_scatter(b2, b4):
        pltpu.make_async_copy(rowsv[b2], acc.at[dstv[b4]], ssem).wait()

    stage(0, 0, 0)
    fire_gather(0)
    stage(1, 1, 1)
    wait_gather(0)
    fire_gather(1)
    conv(0)
    fire_scatter(0, 0)
    stage(2, 0, 2)
    wait_gather(1)
    fire_gather(0)
    conv(1)
    fire_scatter(1, 1)
    stage(3, 1, 3)
    wait_gather(0)
    fire_gather(1)
    wait_scatter(0, 0)
    conv(0)
    fire_scatter(0, 2)
    stage(4, 0, 0)
    wait_gather(1)
    fire_gather(0)
    wait_scatter(1, 1)
    conv(1)
    fire_scatter(1, 3)

    def quad(i, carry):
        for b in range(4):
            k = 4 * i + b
            b2 = b % 2
            b2n = 1 - b2
            stage(k + 1, b2n, (b + 1) % 4)
            wait_gather(b2)
            fire_gather(b2n)
            wait_scatter(b2, (b + 2) % 4)
            conv(b2)
            fire_scatter(b2, b)
        return carry

    lax.fori_loop(1, (_NCH - 1) // 4, quad, 0)

    wait_gather(0)
    wait_scatter(0, 2)
    conv(0)
    fire_scatter(0, 0)
    wait_scatter(1, 3)
    wait_scatter(0, 0)

    plsc.subcore_barrier()
    _writeback(acc, sums_out, c, s, r0)


def _cnt_body(dstr, cnt_out, onesv, dstv0, dstv1, dstvt, zbuf, acc, ssem):
    c = lax.axis_index("c")
    s = lax.axis_index("s")
    r0 = s * _RPT

    _zero_acc(zbuf, acc, r0, s)
    one16 = jnp.ones((16,), jnp.float32)

    def orow(i, carry):
        for j in range(_HW // 16):
            onesv[i, pl.ds(j * 16, 16)] = one16
        return carry

    lax.fori_loop(0, _CC, orow, 0)
    plsc.subcore_barrier()

    ec0 = c * (_E // _NSC) + s * _CEPT
    dstv = (dstv0, dstv1)
    nfull = _CEPT // _CC
    rem = _CEPT - nfull * _CC

    pltpu.sync_copy(dstr.at[pl.ds(ec0, _CC)], dstv[0])

    def pair(i, carry):
        for b in range(2):
            k = 2 * i + b
            bn = 1 - b
            pltpu.async_copy(onesv, acc.at[dstv[b]], ssem, add=True)
            pltpu.sync_copy(dstr.at[pl.ds(ec0 + (k + 1) * _CC, _CC)], dstv[bn])
            pltpu.make_async_copy(onesv, acc.at[dstv[b]], ssem).wait()
        return carry

    lax.fori_loop(0, (nfull - 1) // 2, pair, 0)
    pltpu.sync_copy(onesv, acc.at[dstv[0]], add=True)
    if rem:
        pltpu.sync_copy(dstr.at[pl.ds(ec0 + nfull * _CC, rem)], dstvt)
        pltpu.sync_copy(onesv.at[pl.ds(0, rem), :], acc.at[dstvt], add=True)

    plsc.subcore_barrier()
    _writeback(acc, cnt_out, c, s, r0)


_sc_mesh = plsc.VectorSubcoreMesh(core_axis_name="c", subcore_axis_name="s")

_agg = pl.kernel(
    _agg_body,
    out_type=jax.ShapeDtypeStruct((_NSC, _N, _HW), jnp.float32),
    mesh=_sc_mesh,
    scratch_types=(
        pltpu.VMEM((_C,), jnp.int32),
        pltpu.VMEM((_C,), jnp.int32),
        pltpu.VMEM((_C,), jnp.int32),
        pltpu.VMEM((_C,), jnp.int32),
        pltpu.VMEM((_C,), jnp.int32),
        pltpu.VMEM((_C,), jnp.int32),
        pltpu.VMEM((_C, _HW // 2), jnp.int32),
        pltpu.VMEM((_C, _HW // 2), jnp.int32),
        pltpu.VMEM((_C, _HW), jnp.float32),
        pltpu.VMEM((_C, _HW), jnp.float32),
        pltpu.VMEM((16, _HW), jnp.float32),
        pltpu.VMEM_SHARED((_N, _HW), jnp.float32),
        pltpu.SemaphoreType.DMA,
        pltpu.SemaphoreType.DMA,
    ),
    compiler_params=pltpu.CompilerParams(use_tc_tiling_on_sc=False),
    name="sage_agg",
)

_cnt_kernel = pl.kernel(
    _cnt_body,
    out_type=jax.ShapeDtypeStruct((_NSC, _N, _HW), jnp.float32),
    mesh=_sc_mesh,
    scratch_types=(
        pltpu.VMEM((_CC, _HW), jnp.float32),
        pltpu.VMEM((_CC,), jnp.int32),
        pltpu.VMEM((_CC,), jnp.int32),
        pltpu.VMEM((8,), jnp.int32),
        pltpu.VMEM((16, _HW), jnp.float32),
        pltpu.VMEM_SHARED((_N, _HW), jnp.float32),
        pltpu.SemaphoreType.DMA,
    ),
    name="sage_cnt",
)


_BN = 2000


def _bdot(a, b):
    return jnp.dot(a.astype(jnp.bfloat16), b.astype(jnp.bfloat16),
                   preferred_element_type=jnp.float32)


def _tc1_body(x, s0, s1, c0, c1, wl0a, wl0b, wr0, wl1, bl0, h_out, t_out):
    cnt = c0[:, 0:1] + c1[:, 0:1]
    inv = 1.0 / jnp.maximum(cnt, 1.0)
    h = _bdot(s0[...] * inv, wl0a[...])
    h += _bdot(s1[...] * inv, wl0b[...])
    h += _bdot(x[...], wr0[...])
    h += bl0[...]
    h = jnp.maximum(h, 0.0)
    h_out[...] = h
    t_out[...] = _bdot(h, wl1[...]).astype(jnp.bfloat16)


def _tc2_body(h, s0, s1, c0, c1, wr1, bl1, o_out):
    cnt = c0[:, 0:1] + c1[:, 0:1]
    inv = 1.0 / jnp.maximum(cnt, 1.0)
    m = jnp.concatenate([s0[...] * inv, s1[...] * inv], axis=1)
    o = m + _bdot(h[...], wr1[...])
    o += bl1[...]
    o_out[...] = jnp.maximum(o, 0.0)


def _row_block(width):
    return pl.BlockSpec((_BN, width), lambda i: (i, 0))


def _full_block(shape):
    return pl.BlockSpec(shape, lambda i: tuple(0 for _ in shape))


_tc1 = pl.pallas_call(
    _tc1_body,
    grid=(_N // _BN,),
    in_specs=[
        _row_block(256),
        _row_block(_HW),
        _row_block(_HW),
        _row_block(_HW),
        _row_block(_HW),
        _full_block((_HW, 512)),
        _full_block((_HW, 512)),
        _full_block((256, 512)),
        _full_block((512, 256)),
        _full_block((1, 512)),
    ],
    out_specs=[_row_block(512), _row_block(256)],
    out_shape=[
        jax.ShapeDtypeStruct((_N, 512), jnp.float32),
        jax.ShapeDtypeStruct((_N, 256), jnp.bfloat16),
    ],
)

_tc2 = pl.pallas_call(
    _tc2_body,
    grid=(_N // _BN,),
    in_specs=[
        _row_block(512),
        _row_block(_HW),
        _row_block(_HW),
        _row_block(_HW),
        _row_block(_HW),
        _full_block((512, 256)),
        _full_block((1, 256)),
    ],
    out_specs=_row_block(256),
    out_shape=jax.ShapeDtypeStruct((_N, 256), jnp.float32),
)


_PERM = np.array(
    [32 * (i // 32) + (2 * (i % 32) if (i % 32) < 16 else 2 * ((i % 32) - 16) + 1)
     for i in range(_HW)], dtype=np.int32)
_PINV = np.argsort(_PERM).astype(np.int32)
_CM = np.array([(j // _HW) * _HW + _PINV[j % _HW] for j in range(2 * _HW)],
               dtype=np.int32)


@jax.jit
def kernel(x, edge_index, W_l0, b_l0, W_r0, W_l1, b_l1, W_r1):
    src = edge_index[0]
    dst = edge_index[1]

    cnt = _cnt_kernel(dst)
    x2 = lax.bitcast_convert_type(
        x.astype(jnp.bfloat16).reshape(2 * _N, _HW // 2, 2),
        jnp.int32).reshape(2 * _N, _HW // 2)
    sums0 = _agg(x2, src, dst)
    h, t = _tc1(x, sums0[0], sums0[1], cnt[0], cnt[1],
                W_l0[:_HW][_PERM], W_l0[_HW:][_PERM], W_r0, W_l1[:, _CM],
                b_l0.reshape(1, -1))
    t2 = lax.bitcast_convert_type(
        t.reshape(2 * _N, _HW // 2, 2), jnp.int32).reshape(2 * _N, _HW // 2)
    sums1 = _agg(t2, src, dst)
    out = _tc2(h, sums1[0], sums1[1], cnt[0], cnt[1],
               W_r1, b_l1.reshape(1, -1))
    return out


# speedup vs baseline: 5.4808x; 5.4808x over previous
import jax
import jax.numpy as jnp
from jax import lax
from jax.experimental import pallas as pl
from jax.experimental.pallas import tpu as pltpu
from jax.experimental.pallas import tpu_sc as plsc

_N = 10000
_E = 160000
_NP = 10240
_HW = 128
_NSC = 2
_NTILE = 16
_RPT = _NP // _NTILE
_RPT15 = _N - (_NTILE - 1) * _RPT
_EPT = _E // _NTILE
_C = 80
_NCH = _EPT // _C
_CEPT = _E // (_NSC * _NTILE)
_CC = 128


def _zero_acc(zbuf, acc, r0, s):
    zero16 = jnp.zeros((16,), jnp.float32)

    def zrow(i, carry):
        for j in range(_HW // 16):
            zbuf[i, pl.ds(j * 16, 16)] = zero16
        return carry

    lax.fori_loop(0, 16, zrow, 0)

    def zacc(b, carry):
        pltpu.sync_copy(zbuf, acc.at[pl.ds(r0 + b * 16, 16), :])
        return carry

    lax.fori_loop(0, jnp.where(s == _NTILE - 1, _RPT15 // 16, _RPT // 16),
                  zacc, 0)


def _writeback(acc, out, c, s, r0):
    @pl.when(s < _NTILE - 1)
    def _():
        pltpu.sync_copy(acc.at[pl.ds(r0, _RPT), :],
                        out.at[c, pl.ds(r0, _RPT), :])

    @pl.when(s == _NTILE - 1)
    def _():
        pltpu.sync_copy(acc.at[pl.ds(r0, _RPT15), :],
                        out.at[c, pl.ds(r0, _RPT15), :])


def _agg_body(x2, srcr, dstr, sums_out,
              idxv0, idxv1, idx2v0, idx2v1, dstv0, dstv1,
              rowsv0, rowsv1, zbuf, acc, gsem):
    c = lax.axis_index("c")
    s = lax.axis_index("s")
    r0 = s * _RPT

    _zero_acc(zbuf, acc, r0, s)
    plsc.subcore_barrier()

    idxv = (idxv0, idxv1)
    idx2v = (idx2v0, idx2v1)
    dstv = (dstv0, dstv1)
    rowsv = (rowsv0, rowsv1)

    e0 = s * _EPT

    def stage(k, b):
        eo = e0 + k * _C
        pltpu.sync_copy(srcr.at[pl.ds(eo, _C)], idxv[b])
        pltpu.sync_copy(dstr.at[pl.ds(eo, _C)], dstv[b])
        for j in range(_C // 16):
            v = idxv[b][pl.ds(j * 16, 16)]
            idx2v[b][pl.ds(j * 16, 16)] = v * 2 + c

    stage(0, 0)
    pltpu.async_copy(x2.at[idx2v[0]], rowsv[0], gsem)

    def pair(i, carry):
        for b in range(2):
            k = 2 * i + b
            bn = 1 - b
            stage(k + 1, bn)
            pltpu.make_async_copy(x2.at[idx2v[b]], rowsv[b], gsem).wait()
            pltpu.async_copy(x2.at[idx2v[bn]], rowsv[bn], gsem)
            pltpu.sync_copy(rowsv[b], acc.at[dstv[b]], add=True)
        return carry

    lax.fori_loop(0, (_NCH - 1) // 2, pair, 0)
    pltpu.make_async_copy(x2.at[idx2v[0]], rowsv[0], gsem).wait()
    pltpu.sync_copy(rowsv[0], acc.at[dstv[0]], add=True)

    plsc.subcore_barrier()
    _writeback(acc, sums_out, c, s, r0)


def _cnt_body(dstr, cnt_out, onesv, dstv0, dstv1, dstvt, zbuf, acc, ssem):
    c = lax.axis_index("c")
    s = lax.axis_index("s")
    r0 = s * _RPT

    _zero_acc(zbuf, acc, r0, s)
    one16 = jnp.ones((16,), jnp.float32)

    def orow(i, carry):
        for j in range(_HW // 16):
            onesv[i, pl.ds(j * 16, 16)] = one16
        return carry

    lax.fori_loop(0, _CC, orow, 0)
    plsc.subcore_barrier()

    ec0 = c * (_E // _NSC) + s * _CEPT
    dstv = (dstv0, dstv1)
    nfull = _CEPT // _CC
    rem = _CEPT - nfull * _CC

    pltpu.sync_copy(dstr.at[pl.ds(ec0, _CC)], dstv[0])

    def pair(i, carry):
        for b in range(2):
            k = 2 * i + b
            bn = 1 - b
            pltpu.async_copy(onesv, acc.at[dstv[b]], ssem, add=True)
            pltpu.sync_copy(dstr.at[pl.ds(ec0 + (k + 1) * _CC, _CC)], dstv[bn])
            pltpu.make_async_copy(onesv, acc.at[dstv[b]], ssem).wait()
        return carry

    lax.fori_loop(0, (nfull - 1) // 2, pair, 0)
    pltpu.sync_copy(onesv, acc.at[dstv[0]], add=True)
    if rem:
        pltpu.sync_copy(dstr.at[pl.ds(ec0 + nfull * _CC, rem)], dstvt)
        pltpu.sync_copy(onesv.at[pl.ds(0, rem), :], acc.at[dstvt], add=True)

    plsc.subcore_barrier()
    _writeback(acc, cnt_out, c, s, r0)


_sc_mesh = plsc.VectorSubcoreMesh(core_axis_name="c", subcore_axis_name="s")

_agg = pl.kernel(
    _agg_body,
    out_type=jax.ShapeDtypeStruct((_NSC, _N, _HW), jnp.float32),
    mesh=_sc_mesh,
    scratch_types=(
        pltpu.VMEM((_C,), jnp.int32),
        pltpu.VMEM((_C,), jnp.int32),
        pltpu.VMEM((_C,), jnp.int32),
        pltpu.VMEM((_C,), jnp.int32),
        pltpu.VMEM((_C,), jnp.int32),
        pltpu.VMEM((_C,), jnp.int32),
        pltpu.VMEM((_C, _HW), jnp.float32),
        pltpu.VMEM((_C, _HW), jnp.float32),
        pltpu.VMEM((16, _HW), jnp.float32),
        pltpu.VMEM_SHARED((_NP, _HW), jnp.float32),
        pltpu.SemaphoreType.DMA,
    ),
    name="sage_agg",
)

_cnt_kernel = pl.kernel(
    _cnt_body,
    out_type=jax.ShapeDtypeStruct((_NSC, _N, _HW), jnp.float32),
    mesh=_sc_mesh,
    scratch_types=(
        pltpu.VMEM((_CC, _HW), jnp.float32),
        pltpu.VMEM((_CC,), jnp.int32),
        pltpu.VMEM((_CC,), jnp.int32),
        pltpu.VMEM((8,), jnp.int32),
        pltpu.VMEM((16, _HW), jnp.float32),
        pltpu.VMEM_SHARED((_NP, _HW), jnp.float32),
        pltpu.SemaphoreType.DMA,
    ),
    name="sage_cnt",
)


_BN = 2000


def _bdot(a, b):
    return jnp.dot(a.astype(jnp.bfloat16), b.astype(jnp.bfloat16),
                   preferred_element_type=jnp.float32)


def _tc1_body(x, s0, s1, c0, c1, wl0a, wl0b, wr0, wl1, bl0, h_out, t_out):
    cnt = c0[:, 0:1] + c1[:, 0:1]
    inv = 1.0 / jnp.maximum(cnt, 1.0)
    h = _bdot(s0[...] * inv, wl0a[...])
    h += _bdot(s1[...] * inv, wl0b[...])
    h += _bdot(x[...], wr0[...])
    h += bl0[...]
    h = jnp.maximum(h, 0.0)
    h_out[...] = h.astype(jnp.bfloat16)
    t_out[...] = _bdot(h, wl1[...])


def _tc2_body(h, s0, s1, c0, c1, wr1, bl1, o_out):
    cnt = c0[:, 0:1] + c1[:, 0:1]
    inv = 1.0 / jnp.maximum(cnt, 1.0)
    m = jnp.concatenate([s0[...] * inv, s1[...] * inv], axis=1)
    o = m + _bdot(h[...], wr1[...])
    o += bl1[...]
    o_out[...] = jnp.maximum(o, 0.0)


def _row_block(width):
    return pl.BlockSpec((_BN, width), lambda i: (i, 0))


def _full_block(shape):
    return pl.BlockSpec(shape, lambda i: tuple(0 for _ in shape))


_tc1 = pl.pallas_call(
    _tc1_body,
    grid=(_N // _BN,),
    in_specs=[
        _row_block(256),
        _row_block(_HW),
        _row_block(_HW),
        _row_block(_HW),
        _row_block(_HW),
        _full_block((_HW, 512)),
        _full_block((_HW, 512)),
        _full_block((256, 512)),
        _full_block((512, 256)),
        _full_block((1, 512)),
    ],
    out_specs=[_row_block(512), _row_block(256)],
    out_shape=[
        jax.ShapeDtypeStruct((_N, 512), jnp.bfloat16),
        jax.ShapeDtypeStruct((_N, 256), jnp.float32),
    ],
)

_tc2 = pl.pallas_call(
    _tc2_body,
    grid=(_N // _BN,),
    in_specs=[
        _row_block(512),
        _row_block(_HW),
        _row_block(_HW),
        _row_block(_HW),
        _row_block(_HW),
        _full_block((512, 256)),
        _full_block((1, 256)),
    ],
    out_specs=_row_block(256),
    out_shape=jax.ShapeDtypeStruct((_N, 256), jnp.float32),
)


@jax.jit
def kernel(x, edge_index, W_l0, b_l0, W_r0, W_l1, b_l1, W_r1):
    src = edge_index[0]
    dst = edge_index[1]

    cnt = _cnt_kernel(dst)
    x2 = x.reshape(2 * _N, _HW)
    sums0 = _agg(x2, src, dst)
    h, t = _tc1(x, sums0[0], sums0[1], cnt[0], cnt[1],
                W_l0[:_HW], W_l0[_HW:], W_r0, W_l1, b_l0.reshape(1, -1))
    t2 = t.reshape(2 * _N, _HW)
    sums1 = _agg(t2, src, dst)
    out = _tc2(h, sums1[0], sums1[1], cnt[0], cnt[1],
               W_r1, b_l1.reshape(1, -1))
    return out
